# baseline (speedup 1.0000x reference)
import functools

import jax
import jax.numpy as jnp
from jax import lax
from jax.experimental import pallas as pl
from jax.experimental.pallas import tpu as pltpu
from jax.experimental.pallas import tpu_sc as plsc

F32 = jnp.float32
BF16 = jnp.bfloat16
I32 = jnp.int32

T, D, E, R = 4096, 1024, 256, 64
B, K, G, M, N = 64, 2, 8, 32, 100000
S = K * G * M
KG = K * G
NW = 32
CHUNK = 128
NB = 8
NSPLIT = 2
BH = B // NSPLIT
NSTEP = BH // NB
HB = T // (NSTEP * NSPLIT)


_SC_ROWS = (BH * S) // NW
_SC_NCHUNK = _SC_ROWS // CHUNK


def _sc_gather_body(nbr_hbm, mem_hbm, embs_out, idx_v, buf_v, sem):
    wid = lax.axis_index("s") * 2 + lax.axis_index("c")
    for t in range(_SC_NCHUNK):
        base = wid * _SC_ROWS + t * CHUNK
        pltpu.sync_copy(nbr_hbm.at[pl.ds(base, CHUNK)], idx_v)
        pltpu.async_copy(mem_hbm.at[idx_v], buf_v, sem).wait()
        pltpu.sync_copy(buf_v, embs_out.at[pl.ds(base, CHUNK)])


@functools.cache
def _sc_gather_fn():
    mesh = plsc.VectorSubcoreMesh(core_axis_name="c", subcore_axis_name="s")
    return pl.kernel(
        _sc_gather_body,
        mesh=mesh,
        out_type=jax.ShapeDtypeStruct((BH * S, E), F32),
        scratch_types=[
            pltpu.VMEM((CHUNK,), I32),
            pltpu.VMEM((CHUNK, E), F32),
            pltpu.SemaphoreType.DMA,
        ],
    )


def _sc_gather(nbr_flat, memory_nodes):
    return _sc_gather_fn()(nbr_flat, memory_nodes)


def _k1_body(cat_sm, hid_ref, wq_ref, bq_ref, wcls_ref, bcls_ref,
             q_out, relp_out, rows_v, sem):
    for i in range(2 * B):
        pltpu.make_async_copy(hid_ref.at[pl.ds(cat_sm[i], 1)],
                              rows_v.at[pl.ds(i, 1)], sem).start()
    for i in range(2 * B):
        pltpu.make_async_copy(hid_ref.at[pl.ds(cat_sm[i], 1)],
                              rows_v.at[pl.ds(i, 1)], sem).wait()
    rows = rows_v[...]
    g = rows[0:B]
    r = rows[B:2 * B]
    q_out[...] = jnp.tanh(
        jnp.dot(g, wq_ref[...], preferred_element_type=F32) + bq_ref[...])
    logits = jnp.dot(r, wcls_ref[...], preferred_element_type=F32) + bcls_ref[...]
    mx = jnp.max(logits, axis=1, keepdims=True)
    e = jnp.exp(logits - mx)
    relp_out[...] = e / jnp.sum(e, axis=1, keepdims=True)


def _k1(cat_idx, hidden_states, W_q, b_q2, W_cls, b_cls2):
    return pl.pallas_call(
        _k1_body,
        in_specs=[
            pl.BlockSpec(memory_space=pltpu.MemorySpace.SMEM),
            pl.BlockSpec(memory_space=pltpu.MemorySpace.HBM),
            pl.BlockSpec((D, D), lambda: (0, 0)),
            pl.BlockSpec((1, D), lambda: (0, 0)),
            pl.BlockSpec((D, R), lambda: (0, 0)),
            pl.BlockSpec((1, R), lambda: (0, 0)),
        ],
        out_shape=[
            jax.ShapeDtypeStruct((B, D), F32),
            jax.ShapeDtypeStruct((B, R), F32),
        ],
        scratch_shapes=[
            pltpu.VMEM((2 * B, D), F32),
            pltpu.SemaphoreType.DMA,
        ],
    )(cat_idx, hidden_states, W_q, b_q2, W_cls, b_cls2)


SB = NB * S


def _k2_body(embs_ref, q_ref, relp_ref, grp_ref, wn_ref, bn_ref, wg_ref,
             bg_ref, expTBD_ref, konehBD_ref, grpA_ref, bmask_ref, blk64_ref,
             hid_ref, row_out, hid_out):
    hid_out[...] = hid_ref[...]
    embs = embs_ref[...]

    proj = jnp.tanh(
        jnp.dot(embs.astype(BF16), wn_ref[...],
                preferred_element_type=F32) + bn_ref[...]).astype(BF16)

    q2 = q_ref[:, 0, :].astype(BF16)
    dots = lax.dot_general(proj, q2, (((1,), (1,)), ((), ())),
                           preferred_element_type=F32)
    dt = jnp.transpose(dots)

    bmask = bmask_ref[...]
    neg = 1.0 - bmask
    rowmax = jnp.max(dt - neg * 1e30, axis=1, keepdims=True)
    e1 = jnp.exp(jnp.minimum(dt - rowmax, 0.0)) * bmask
    gsum = jnp.dot(e1, expTBD_ref[...], preferred_element_type=F32)
    denom = lax.dot_general(gsum, expTBD_ref[...], (((1,), (1,)), ((), ())),
                            preferred_element_type=F32) + neg
    attn = e1 / denom

    grp = grp_ref[0]
    oneh = (grpA_ref[...] == grp).astype(F32)
    relp2 = relp_ref[:, 0, :]
    relpBD = jnp.concatenate([relp2] * NB, axis=1) * blk64_ref[...]
    p16 = jnp.dot(relpBD, oneh, preferred_element_type=F32)
    p_slot = lax.dot_general(p16, expTBD_ref[...], (((1,), (1,)), ((), ())),
                             preferred_element_type=F32)

    e2 = jnp.exp(attn * p_slot * 10.0) * bmask
    ksum = jnp.dot(e2, konehBD_ref[...], preferred_element_type=F32)
    denom2 = lax.dot_general(ksum, konehBD_ref[...], (((1,), (1,)), ((), ())),
                             preferred_element_type=F32) + neg
    coef = (e2 / denom2 * (1.0 / (G * M * K))).astype(BF16)

    mask = (embs[:, 0:1] != 0.0).astype(BF16)
    membs = embs.astype(BF16) * mask
    asc = jnp.dot(coef, membs, preferred_element_type=F32)
    outp = jnp.tanh(
        jnp.dot(asc.astype(BF16), wg_ref[...],
                preferred_element_type=F32) + bg_ref[...])
    row_out[...] = outp.reshape(NB, 1, D)


def _k2_chunk(c, embs, q3, relp3, grp3, W_nodes, b_n2, W_gnn, b_g2, expTBD,
              konehBD, grpA, bmaskC, blk64C, hidden_states, prev_newhid=None):
    off = c * NSTEP
    in_specs = [
        pl.BlockSpec((NB * S, E), lambda b: (b, 0)),
        pl.BlockSpec((NB, 1, D), lambda b: (b, 0, 0)),
        pl.BlockSpec((NB, 1, R), lambda b: (b, 0, 0)),
        pl.BlockSpec((1, 1, NB * KG), lambda b: (b, 0, 0)),
        pl.BlockSpec((E, D), lambda b: (0, 0)),
        pl.BlockSpec((1, D), lambda b: (0, 0)),
        pl.BlockSpec((E, D), lambda b: (0, 0)),
        pl.BlockSpec((1, D), lambda b: (0, 0)),
        pl.BlockSpec((SB, NB * KG), lambda b: (0, 0)),
        pl.BlockSpec((SB, NB * K), lambda b: (0, 0)),
        pl.BlockSpec((S, NB * KG), lambda b: (0, 0)),
        pl.BlockSpec((NB, SB), lambda b: (0, 0)),
        pl.BlockSpec((NB, S), lambda b: (0, 0)),
        pl.BlockSpec((HB, D), lambda b, off=off: (b + off, 0)),
    ]
    args = [embs, q3, relp3, grp3, W_nodes, b_n2, W_gnn, b_g2, expTBD,
            konehBD, grpA, bmaskC, blk64C, hidden_states]
    aliases = {}
    body = _k2_body
    if prev_newhid is not None:
        in_specs.append(pl.BlockSpec(memory_space=pltpu.MemorySpace.HBM))
        args.append(prev_newhid)
        aliases = {14: 1}

        def body(*refs):
            _k2_body(*refs[:14], *refs[15:])

    return pl.pallas_call(
        body,
        grid=(NSTEP,),
        in_specs=in_specs,
        out_specs=[
            pl.BlockSpec((NB, 1, D), lambda b: (b, 0, 0)),
            pl.BlockSpec((HB, D), lambda b, off=off: (b + off, 0)),
        ],
        out_shape=[
            jax.ShapeDtypeStruct((BH, 1, D), F32),
            jax.ShapeDtypeStruct((T, D), F32),
        ],
        input_output_aliases=aliases,
        compiler_params=pltpu.CompilerParams(
            dimension_semantics=("arbitrary",)),
    )(*args)


def _k3_body(hid_ref, gnn_sm, gcol_ref, grow_ref, upd_ref, out_ref,
             rows_v, sem):
    del hid_ref
    dup = (gcol_ref[...] == grow_ref[...]).astype(F32)
    upd = jnp.dot(dup, upd_ref[...], preferred_element_type=F32)
    for b in range(B):
        pltpu.make_async_copy(out_ref.at[pl.ds(gnn_sm[b], 1)],
                              rows_v.at[pl.ds(b, 1)], sem).start()
    for b in range(B):
        pltpu.make_async_copy(out_ref.at[pl.ds(gnn_sm[b], 1)],
                              rows_v.at[pl.ds(b, 1)], sem).wait()
    rows_v[...] = rows_v[...] + upd
    for b in range(B):
        pltpu.make_async_copy(rows_v.at[pl.ds(b, 1)],
                              out_ref.at[pl.ds(gnn_sm[b], 1)], sem).start()
    for b in range(B):
        pltpu.make_async_copy(rows_v.at[pl.ds(b, 1)],
                              out_ref.at[pl.ds(gnn_sm[b], 1)], sem).wait()


def _k3(new_hidden, gnn_i32, gnn_col, gnn_row, upd_rows):
    return pl.pallas_call(
        _k3_body,
        in_specs=[
            pl.BlockSpec(memory_space=pltpu.MemorySpace.HBM),
            pl.BlockSpec(memory_space=pltpu.MemorySpace.SMEM),
            pl.BlockSpec((B, 1), lambda: (0, 0)),
            pl.BlockSpec((1, B), lambda: (0, 0)),
            pl.BlockSpec((B, D), lambda: (0, 0)),
        ],
        out_specs=pl.BlockSpec(memory_space=pltpu.MemorySpace.HBM),
        out_shape=jax.ShapeDtypeStruct((T, D), F32),
        input_output_aliases={0: 0},
        scratch_shapes=[
            pltpu.VMEM((B, D), F32),
            pltpu.SemaphoreType.DMA,
        ],
    )(new_hidden, gnn_i32, gnn_col, gnn_row, upd_rows)


def kernel(hidden_states, memory_nodes, gnn_idx, rel_idx, neighbor_idx,
           group_rel_ids, W_cls, b_cls, W_q, b_q, W_nodes, b_nodes, W_gnn,
           b_gnn):
    nbr_flat = neighbor_idx.reshape(-1).astype(I32)
    cat_idx = jnp.concatenate([gnn_idx, rel_idx]).astype(I32)
    sb = jnp.arange(SB, dtype=I32)
    expTBD = (sb[:, None] // M == jnp.arange(NB * KG, dtype=I32)[None, :]
              ).astype(F32)
    konehBD = (sb[:, None] // (G * M) == jnp.arange(NB * K, dtype=I32)[None, :]
               ).astype(F32)
    p_ = jnp.arange(S, dtype=I32)
    c_ = jnp.arange(NB * KG, dtype=I32)
    grpA = jnp.where(p_[:, None] // R == c_[None, :] // KG,
                     p_[:, None] % R, -1).astype(I32)
    j_ = jnp.arange(NB, dtype=I32)
    bmaskC = (sb[None, :] // S == j_[:, None]).astype(F32)
    blk64C = (p_[None, :] // R == j_[:, None]).astype(F32)

    q, rel_prob = _k1(cat_idx, hidden_states, W_q, b_q.reshape(1, D), W_cls,
                      b_cls.reshape(1, R))
    q3 = q.reshape(B, 1, D)
    relp3 = rel_prob.reshape(B, 1, R)
    grp3 = group_rel_ids.reshape(B // NB, 1, NB * KG).astype(I32)
    wn = W_nodes.astype(BF16)
    wg = W_gnn.astype(BF16)
    bn = b_nodes.reshape(1, D)
    bg = b_gnn.reshape(1, D)

    embs_parts = [
        _sc_gather(nbr_flat[c * BH * S:(c + 1) * BH * S], memory_nodes)
        for c in range(NSPLIT)
    ]
    rows_parts = []
    new_hidden = None
    for c in range(NSPLIT):
        rows_c, new_hidden = _k2_chunk(
            c, embs_parts[c], q3[c * BH:(c + 1) * BH],
            relp3[c * BH:(c + 1) * BH],
            grp3[c * NSTEP:(c + 1) * NSTEP], wn, bn, wg, bg, expTBD,
            konehBD, grpA, bmaskC, blk64C, hidden_states, new_hidden)
        rows_parts.append(rows_c.reshape(BH, D))
    out_rows = jnp.concatenate(rows_parts, axis=0)

    gnn_i32 = gnn_idx.astype(I32)
    return _k3(new_hidden, gnn_i32, gnn_i32.reshape(B, 1),
               gnn_i32.reshape(1, B), out_rows)


# speedup vs baseline: 1.0309x; 1.0309x over previous
import functools

import jax
import jax.numpy as jnp
from jax import lax
from jax.experimental import pallas as pl
from jax.experimental.pallas import tpu as pltpu
from jax.experimental.pallas import tpu_sc as plsc

F32 = jnp.float32
BF16 = jnp.bfloat16
I32 = jnp.int32

T, D, E, R = 4096, 1024, 256, 64
B, K, G, M, N = 64, 2, 8, 32, 100000
S = K * G * M
KG = K * G
NW = 32
CHUNK = 128
NB = 8
NSPLIT = 2
BH = B // NSPLIT
NSTEP = BH // NB
HB = T // (NSTEP * NSPLIT)


_SC_ROWS = (BH * S) // NW
_SC_NCHUNK = _SC_ROWS // CHUNK


def _sc_gather_body(nbr_hbm, mem_hbm, embs_out, idx_v, buf_v, sem):
    wid = lax.axis_index("s") * 2 + lax.axis_index("c")
    for t in range(_SC_NCHUNK):
        base = wid * _SC_ROWS + t * CHUNK
        pltpu.sync_copy(nbr_hbm.at[pl.ds(base, CHUNK)], idx_v)
        pltpu.async_copy(mem_hbm.at[idx_v], buf_v, sem).wait()
        pltpu.sync_copy(buf_v, embs_out.at[pl.ds(base, CHUNK)])


@functools.cache
def _sc_gather_fn():
    mesh = plsc.VectorSubcoreMesh(core_axis_name="c", subcore_axis_name="s")
    return pl.kernel(
        _sc_gather_body,
        mesh=mesh,
        out_type=jax.ShapeDtypeStruct((BH * S, E), F32),
        scratch_types=[
            pltpu.VMEM((CHUNK,), I32),
            pltpu.VMEM((CHUNK, E), F32),
            pltpu.SemaphoreType.DMA,
        ],
    )


def _sc_gather(nbr_flat, memory_nodes):
    return _sc_gather_fn()(nbr_flat, memory_nodes)


def _k1_body(cat_sm, hid_ref, wq_ref, bq_ref, wcls_ref, bcls_ref,
             q_out, relp_out, rows_v, sem):
    for i in range(2 * B):
        pltpu.make_async_copy(hid_ref.at[pl.ds(cat_sm[i], 1)],
                              rows_v.at[pl.ds(i, 1)], sem).start()
    for i in range(2 * B):
        pltpu.make_async_copy(hid_ref.at[pl.ds(cat_sm[i], 1)],
                              rows_v.at[pl.ds(i, 1)], sem).wait()
    rows = rows_v[...]
    g = rows[0:B]
    r = rows[B:2 * B]
    q_out[...] = jnp.tanh(
        jnp.dot(g, wq_ref[...], preferred_element_type=F32) + bq_ref[...])
    logits = jnp.dot(r, wcls_ref[...], preferred_element_type=F32) + bcls_ref[...]
    mx = jnp.max(logits, axis=1, keepdims=True)
    e = jnp.exp(logits - mx)
    relp_out[...] = e / jnp.sum(e, axis=1, keepdims=True)


def _k1(cat_idx, hidden_states, W_q, b_q2, W_cls, b_cls2):
    return pl.pallas_call(
        _k1_body,
        in_specs=[
            pl.BlockSpec(memory_space=pltpu.MemorySpace.SMEM),
            pl.BlockSpec(memory_space=pltpu.MemorySpace.HBM),
            pl.BlockSpec((D, D), lambda: (0, 0)),
            pl.BlockSpec((1, D), lambda: (0, 0)),
            pl.BlockSpec((D, R), lambda: (0, 0)),
            pl.BlockSpec((1, R), lambda: (0, 0)),
        ],
        out_shape=[
            jax.ShapeDtypeStruct((B, D), F32),
            jax.ShapeDtypeStruct((B, R), F32),
        ],
        scratch_shapes=[
            pltpu.VMEM((2 * B, D), F32),
            pltpu.SemaphoreType.DMA,
        ],
    )(cat_idx, hidden_states, W_q, b_q2, W_cls, b_cls2)


SB = NB * S


def _k2_body(embs_ref, q_ref, relp_ref, grp_ref, wn_ref, bn_ref, wg_ref,
             bg_ref, expTBD_ref, konehBD_ref, grpA_ref, bmask_ref, blk64_ref,
             gmask_ref, hid_ref, row_out, hid_out, copy_sem):
    pltpu.make_async_copy(hid_ref, hid_out, copy_sem).start()
    embs = embs_ref[...]

    proj = jnp.tanh(
        jnp.dot(embs.astype(BF16), wn_ref[...],
                preferred_element_type=F32) + bn_ref[...]).astype(BF16)

    q2 = q_ref[:, 0, :].astype(BF16)
    dots = lax.dot_general(proj, q2, (((1,), (1,)), ((), ())),
                           preferred_element_type=F32)
    dt = jnp.transpose(dots)

    bmask = bmask_ref[...]
    neg = 1.0 - bmask
    rowmax = jnp.max(dt - neg * 1e30, axis=1, keepdims=True)
    e1 = jnp.exp(jnp.minimum(dt - rowmax, 0.0)) * bmask
    gsum = jnp.dot(e1.astype(BF16), expTBD_ref[...],
                   preferred_element_type=F32)

    grp = grp_ref[0]
    oneh = (grpA_ref[...] == grp).astype(BF16)
    relp2 = relp_ref[:, 0, :]
    relpBD = (jnp.concatenate([relp2] * NB, axis=1)
              * blk64_ref[...]).astype(BF16)
    p16 = jnp.dot(relpBD, oneh, preferred_element_type=F32)
    gmask = gmask_ref[...]
    pg = (p16 * 10.0 / (gsum + (1.0 - gmask))).astype(BF16)
    pgs = lax.dot_general(pg, expTBD_ref[...], (((1,), (1,)), ((), ())),
                          preferred_element_type=F32)

    e2 = jnp.exp(e1 * pgs) * bmask
    ksum = jnp.dot(e2.astype(BF16), konehBD_ref[...],
                   preferred_element_type=F32)
    denom2 = lax.dot_general(ksum.astype(BF16), konehBD_ref[...],
                             (((1,), (1,)), ((), ())),
                             preferred_element_type=F32) + neg
    coef = (e2 / denom2 * (1.0 / (G * M * K))).astype(BF16)

    mask = (embs[:, 0:1] != 0.0).astype(BF16)
    membs = embs.astype(BF16) * mask
    asc = jnp.dot(coef, membs, preferred_element_type=F32)
    outp = jnp.tanh(
        jnp.dot(asc.astype(BF16), wg_ref[...],
                preferred_element_type=F32) + bg_ref[...])
    row_out[...] = outp.reshape(NB, 1, D)
    pltpu.make_async_copy(hid_ref, hid_out, copy_sem).wait()


def _k2_chunk(c, embs, q3, relp3, grp3, W_nodes, b_n2, W_gnn, b_g2, expTBD,
              konehBD, grpA, bmaskC, blk64C, gmaskC, hidden_states,
              prev_newhid=None):
    off = c * NSTEP
    in_specs = [
        pl.BlockSpec((NB * S, E), lambda b: (b, 0)),
        pl.BlockSpec((NB, 1, D), lambda b: (b, 0, 0)),
        pl.BlockSpec((NB, 1, R), lambda b: (b, 0, 0)),
        pl.BlockSpec((1, 1, NB * KG), lambda b: (b, 0, 0)),
        pl.BlockSpec((E, D), lambda b: (0, 0)),
        pl.BlockSpec((1, D), lambda b: (0, 0)),
        pl.BlockSpec((E, D), lambda b: (0, 0)),
        pl.BlockSpec((1, D), lambda b: (0, 0)),
        pl.BlockSpec((SB, NB * KG), lambda b: (0, 0)),
        pl.BlockSpec((SB, NB * K), lambda b: (0, 0)),
        pl.BlockSpec((S, NB * KG), lambda b: (0, 0)),
        pl.BlockSpec((NB, SB), lambda b: (0, 0)),
        pl.BlockSpec((NB, S), lambda b: (0, 0)),
        pl.BlockSpec((NB, NB * KG), lambda b: (0, 0)),
        pl.BlockSpec((HB, D), lambda b, off=off: (b + off, 0)),
    ]
    args = [embs, q3, relp3, grp3, W_nodes, b_n2, W_gnn, b_g2, expTBD,
            konehBD, grpA, bmaskC, blk64C, gmaskC, hidden_states]
    aliases = {}
    body = _k2_body
    if prev_newhid is not None:
        in_specs.append(pl.BlockSpec(memory_space=pltpu.MemorySpace.HBM))
        args.append(prev_newhid)
        aliases = {15: 1}

        def body(*refs):
            _k2_body(*refs[:15], *refs[16:])

    return pl.pallas_call(
        body,
        grid=(NSTEP,),
        in_specs=in_specs,
        out_specs=[
            pl.BlockSpec((NB, 1, D), lambda b: (b, 0, 0)),
            pl.BlockSpec((HB, D), lambda b, off=off: (b + off, 0)),
        ],
        out_shape=[
            jax.ShapeDtypeStruct((BH, 1, D), F32),
            jax.ShapeDtypeStruct((T, D), F32),
        ],
        input_output_aliases=aliases,
        scratch_shapes=[pltpu.SemaphoreType.DMA],
        compiler_params=pltpu.CompilerParams(
            dimension_semantics=("arbitrary",)),
    )(*args)


def _k3_body(hid_ref, gnn_sm, gcol_ref, grow_ref, upd_ref, out_ref,
             rows_v, sem):
    del hid_ref
    dup = (gcol_ref[...] == grow_ref[...]).astype(F32)
    upd = jnp.dot(dup, upd_ref[...], preferred_element_type=F32)
    for b in range(B):
        pltpu.make_async_copy(out_ref.at[pl.ds(gnn_sm[b], 1)],
                              rows_v.at[pl.ds(b, 1)], sem).start()
    for b in range(B):
        pltpu.make_async_copy(out_ref.at[pl.ds(gnn_sm[b], 1)],
                              rows_v.at[pl.ds(b, 1)], sem).wait()
    rows_v[...] = rows_v[...] + upd
    for b in range(B):
        pltpu.make_async_copy(rows_v.at[pl.ds(b, 1)],
                              out_ref.at[pl.ds(gnn_sm[b], 1)], sem).start()
    for b in range(B):
        pltpu.make_async_copy(rows_v.at[pl.ds(b, 1)],
                              out_ref.at[pl.ds(gnn_sm[b], 1)], sem).wait()


def _k3(new_hidden, gnn_i32, gnn_col, gnn_row, upd_rows):
    return pl.pallas_call(
        _k3_body,
        in_specs=[
            pl.BlockSpec(memory_space=pltpu.MemorySpace.HBM),
            pl.BlockSpec(memory_space=pltpu.MemorySpace.SMEM),
            pl.BlockSpec((B, 1), lambda: (0, 0)),
            pl.BlockSpec((1, B), lambda: (0, 0)),
            pl.BlockSpec((B, D), lambda: (0, 0)),
        ],
        out_specs=pl.BlockSpec(memory_space=pltpu.MemorySpace.HBM),
        out_shape=jax.ShapeDtypeStruct((T, D), F32),
        input_output_aliases={0: 0},
        scratch_shapes=[
            pltpu.VMEM((B, D), F32),
            pltpu.SemaphoreType.DMA,
        ],
    )(new_hidden, gnn_i32, gnn_col, gnn_row, upd_rows)


def kernel(hidden_states, memory_nodes, gnn_idx, rel_idx, neighbor_idx,
           group_rel_ids, W_cls, b_cls, W_q, b_q, W_nodes, b_nodes, W_gnn,
           b_gnn):
    nbr_flat = neighbor_idx.reshape(-1).astype(I32)
    cat_idx = jnp.concatenate([gnn_idx, rel_idx]).astype(I32)
    sb = jnp.arange(SB, dtype=I32)
    expTBD = (sb[:, None] // M == jnp.arange(NB * KG, dtype=I32)[None, :]
              ).astype(BF16)
    konehBD = (sb[:, None] // (G * M) == jnp.arange(NB * K, dtype=I32)[None, :]
               ).astype(BF16)
    p_ = jnp.arange(S, dtype=I32)
    c_ = jnp.arange(NB * KG, dtype=I32)
    grpA = jnp.where(p_[:, None] // R == c_[None, :] // KG,
                     p_[:, None] % R, -1).astype(I32)
    j_ = jnp.arange(NB, dtype=I32)
    bmaskC = (sb[None, :] // S == j_[:, None]).astype(F32)
    blk64C = (p_[None, :] // R == j_[:, None]).astype(F32)
    gmaskC = (c_[None, :] // KG == j_[:, None]).astype(F32)

    q, rel_prob = _k1(cat_idx, hidden_states, W_q, b_q.reshape(1, D), W_cls,
                      b_cls.reshape(1, R))
    q3 = q.reshape(B, 1, D)
    relp3 = rel_prob.reshape(B, 1, R)
    grp3 = group_rel_ids.reshape(B // NB, 1, NB * KG).astype(I32)
    wn = W_nodes.astype(BF16)
    wg = W_gnn.astype(BF16)
    bn = b_nodes.reshape(1, D)
    bg = b_gnn.reshape(1, D)

    embs_parts = [
        _sc_gather(nbr_flat[c * BH * S:(c + 1) * BH * S], memory_nodes)
        for c in range(NSPLIT)
    ]
    rows_parts = []
    new_hidden = None
    for c in range(NSPLIT):
        rows_c, new_hidden = _k2_chunk(
            c, embs_parts[c], q3[c * BH:(c + 1) * BH],
            relp3[c * BH:(c + 1) * BH],
            grp3[c * NSTEP:(c + 1) * NSTEP], wn, bn, wg, bg, expTBD,
            konehBD, grpA, bmaskC, blk64C, gmaskC, hidden_states, new_hidden)
        rows_parts.append(rows_c.reshape(BH, D))
    out_rows = jnp.concatenate(rows_parts, axis=0)

    gnn_i32 = gnn_idx.astype(I32)
    return _k3(new_hidden, gnn_i32, gnn_i32.reshape(B, 1),
               gnn_i32.reshape(1, B), out_rows)


# speedup vs baseline: 1.1215x; 1.0879x over previous
import functools

import numpy as np

import jax
import jax.numpy as jnp
from jax import lax
from jax.experimental import pallas as pl
from jax.experimental.pallas import tpu as pltpu
from jax.experimental.pallas import tpu_sc as plsc

F32 = jnp.float32
BF16 = jnp.bfloat16
I32 = jnp.int32

T, D, E, R = 4096, 1024, 256, 64
B, K, G, M, N = 64, 2, 8, 32, 100000
S = K * G * M
KG = K * G
NW = 32
CHUNK = 128
NB = 8
NSPLIT = 2
BH = B // NSPLIT
NSTEP = BH // NB
HB = T // (NSTEP * NSPLIT)


_SC_ROWS = (BH * S) // NW
_SC_NCHUNK = _SC_ROWS // CHUNK


def _sc_gather_body(nbr_hbm, mem_hbm, embs_out, idx_v, buf_v, sem):
    wid = lax.axis_index("s") * 2 + lax.axis_index("c")
    for t in range(_SC_NCHUNK):
        base = wid * _SC_ROWS + t * CHUNK
        pltpu.sync_copy(nbr_hbm.at[pl.ds(base, CHUNK)], idx_v)
        pltpu.async_copy(mem_hbm.at[idx_v], buf_v, sem).wait()
        pltpu.sync_copy(buf_v, embs_out.at[pl.ds(base, CHUNK)])


@functools.cache
def _sc_gather_fn():
    mesh = plsc.VectorSubcoreMesh(core_axis_name="c", subcore_axis_name="s")
    return pl.kernel(
        _sc_gather_body,
        mesh=mesh,
        out_type=jax.ShapeDtypeStruct((BH * S, E), F32),
        scratch_types=[
            pltpu.VMEM((CHUNK,), I32),
            pltpu.VMEM((CHUNK, E), F32),
            pltpu.SemaphoreType.DMA,
        ],
    )


def _sc_gather(nbr_flat, memory_nodes):
    return _sc_gather_fn()(nbr_flat, memory_nodes)


def _k1_body(cat_sm, hid_ref, wq_ref, bq_ref, wcls_ref, bcls_ref,
             q_out, relp_out, rows_v, sem):
    for i in range(2 * B):
        pltpu.make_async_copy(hid_ref.at[pl.ds(cat_sm[i], 1)],
                              rows_v.at[pl.ds(i, 1)], sem).start()
    for i in range(2 * B):
        pltpu.make_async_copy(hid_ref.at[pl.ds(cat_sm[i], 1)],
                              rows_v.at[pl.ds(i, 1)], sem).wait()
    rows = rows_v[...]
    g = rows[0:B]
    r = rows[B:2 * B]
    q_out[...] = jnp.tanh(
        jnp.dot(g, wq_ref[...], preferred_element_type=F32) + bq_ref[...])
    logits = jnp.dot(r, wcls_ref[...], preferred_element_type=F32) + bcls_ref[...]
    mx = jnp.max(logits, axis=1, keepdims=True)
    e = jnp.exp(logits - mx)
    relp_out[...] = e / jnp.sum(e, axis=1, keepdims=True)


def _k1(cat_idx, hidden_states, W_q, b_q2, W_cls, b_cls2):
    return pl.pallas_call(
        _k1_body,
        in_specs=[
            pl.BlockSpec(memory_space=pltpu.MemorySpace.SMEM),
            pl.BlockSpec(memory_space=pltpu.MemorySpace.HBM),
            pl.BlockSpec((D, D), lambda: (0, 0)),
            pl.BlockSpec((1, D), lambda: (0, 0)),
            pl.BlockSpec((D, R), lambda: (0, 0)),
            pl.BlockSpec((1, R), lambda: (0, 0)),
        ],
        out_shape=[
            jax.ShapeDtypeStruct((B, D), F32),
            jax.ShapeDtypeStruct((B, R), F32),
        ],
        scratch_shapes=[
            pltpu.VMEM((2 * B, D), F32),
            pltpu.SemaphoreType.DMA,
        ],
    )(cat_idx, hidden_states, W_q, b_q2, W_cls, b_cls2)


SB = NB * S


def _k2_body(embs_ref, q_ref, relp_ref, grp_ref, wn_ref, bn_ref, wg_ref,
             bg_ref, expTBD_ref, konehBD_ref, grpA_ref, bmask_ref, blk64_ref,
             gmask_ref, hid_ref, row_out, hid_out, copy_sem):
    pltpu.make_async_copy(hid_ref, hid_out, copy_sem).start()
    embs = embs_ref[...]

    proj = jnp.tanh(
        jnp.dot(embs.astype(BF16), wn_ref[...],
                preferred_element_type=F32) + bn_ref[...]).astype(BF16)

    q2 = q_ref[:, 0, :].astype(BF16)
    dots = lax.dot_general(proj, q2, (((1,), (1,)), ((), ())),
                           preferred_element_type=F32)
    dt = jnp.transpose(dots)

    bmask = bmask_ref[...]
    neg = 1.0 - bmask
    rowmax = jnp.max(dt - neg * 1e30, axis=1, keepdims=True)
    e1 = jnp.exp(jnp.minimum(dt - rowmax, 0.0)) * bmask
    gsum = jnp.dot(e1.astype(BF16), expTBD_ref[...],
                   preferred_element_type=F32)

    grp = grp_ref[0]
    oneh = (grpA_ref[...] == grp).astype(BF16)
    relp2 = relp_ref[:, 0, :]
    relpBD = (jnp.concatenate([relp2] * NB, axis=1)
              * blk64_ref[...]).astype(BF16)
    p16 = jnp.dot(relpBD, oneh, preferred_element_type=F32)
    gmask = gmask_ref[...]
    pg = (p16 * 10.0 / (gsum + (1.0 - gmask))).astype(BF16)
    pgs = lax.dot_general(pg, expTBD_ref[...], (((1,), (1,)), ((), ())),
                          preferred_element_type=F32)

    e2 = jnp.exp(e1 * pgs) * bmask
    ksum = jnp.dot(e2.astype(BF16), konehBD_ref[...],
                   preferred_element_type=F32)
    denom2 = lax.dot_general(ksum.astype(BF16), konehBD_ref[...],
                             (((1,), (1,)), ((), ())),
                             preferred_element_type=F32) + neg
    coef = (e2 / denom2 * (1.0 / (G * M * K))).astype(BF16)

    mask = (embs[:, 0:1] != 0.0).astype(BF16)
    membs = embs.astype(BF16) * mask
    asc = jnp.dot(coef, membs, preferred_element_type=F32)
    outp = jnp.tanh(
        jnp.dot(asc.astype(BF16), wg_ref[...],
                preferred_element_type=F32) + bg_ref[...])
    row_out[...] = outp.reshape(NB, 1, D)
    pltpu.make_async_copy(hid_ref, hid_out, copy_sem).wait()


def _k2_chunk(c, embs, q3, relp3, grp3, W_nodes, b_n2, W_gnn, b_g2, expTBD,
              konehBD, grpA, bmaskC, blk64C, gmaskC, hidden_states,
              prev_newhid=None):
    off = c * NSTEP
    in_specs = [
        pl.BlockSpec((NB * S, E), lambda b: (b, 0)),
        pl.BlockSpec((NB, 1, D), lambda b: (b, 0, 0)),
        pl.BlockSpec((NB, 1, R), lambda b: (b, 0, 0)),
        pl.BlockSpec((1, 1, NB * KG), lambda b: (b, 0, 0)),
        pl.BlockSpec((E, D), lambda b: (0, 0)),
        pl.BlockSpec((1, D), lambda b: (0, 0)),
        pl.BlockSpec((E, D), lambda b: (0, 0)),
        pl.BlockSpec((1, D), lambda b: (0, 0)),
        pl.BlockSpec((SB, NB * KG), lambda b: (0, 0)),
        pl.BlockSpec((SB, NB * K), lambda b: (0, 0)),
        pl.BlockSpec((S, NB * KG), lambda b: (0, 0)),
        pl.BlockSpec((NB, SB), lambda b: (0, 0)),
        pl.BlockSpec((NB, S), lambda b: (0, 0)),
        pl.BlockSpec((NB, NB * KG), lambda b: (0, 0)),
        pl.BlockSpec((HB, D), lambda b, off=off: (b + off, 0)),
    ]
    args = [embs, q3, relp3, grp3, W_nodes, b_n2, W_gnn, b_g2, expTBD,
            konehBD, grpA, bmaskC, blk64C, gmaskC, hidden_states]
    aliases = {}
    body = _k2_body
    if prev_newhid is not None:
        in_specs.append(pl.BlockSpec(memory_space=pltpu.MemorySpace.HBM))
        args.append(prev_newhid)
        aliases = {15: 1}

        def body(*refs):
            _k2_body(*refs[:15], *refs[16:])

    return pl.pallas_call(
        body,
        grid=(NSTEP,),
        in_specs=in_specs,
        out_specs=[
            pl.BlockSpec((NB, 1, D), lambda b: (b, 0, 0)),
            pl.BlockSpec((HB, D), lambda b, off=off: (b + off, 0)),
        ],
        out_shape=[
            jax.ShapeDtypeStruct((BH, 1, D), F32),
            jax.ShapeDtypeStruct((T, D), F32),
        ],
        input_output_aliases=aliases,
        scratch_shapes=[pltpu.SemaphoreType.DMA],
        compiler_params=pltpu.CompilerParams(
            dimension_semantics=("arbitrary",)),
    )(*args)


def _k3_body(hid_ref, gnn_sm, gcol_ref, grow_ref, u0_ref, u1_ref, out_ref,
             rows_v, sem):
    del hid_ref
    dup = (gcol_ref[...] == grow_ref[...]).astype(F32)
    upd_all = jnp.concatenate([u0_ref[:, 0, :], u1_ref[:, 0, :]], axis=0)
    upd = jnp.dot(dup, upd_all, preferred_element_type=F32)
    for b in range(B):
        pltpu.make_async_copy(out_ref.at[pl.ds(gnn_sm[b], 1)],
                              rows_v.at[pl.ds(b, 1)], sem).start()
    for b in range(B):
        pltpu.make_async_copy(out_ref.at[pl.ds(gnn_sm[b], 1)],
                              rows_v.at[pl.ds(b, 1)], sem).wait()
    rows_v[...] = rows_v[...] + upd
    for b in range(B):
        pltpu.make_async_copy(rows_v.at[pl.ds(b, 1)],
                              out_ref.at[pl.ds(gnn_sm[b], 1)], sem).start()
    for b in range(B):
        pltpu.make_async_copy(rows_v.at[pl.ds(b, 1)],
                              out_ref.at[pl.ds(gnn_sm[b], 1)], sem).wait()


def _k3(new_hidden, gnn_i32, gnn_col, gnn_row, u0, u1):
    return pl.pallas_call(
        _k3_body,
        in_specs=[
            pl.BlockSpec(memory_space=pltpu.MemorySpace.HBM),
            pl.BlockSpec(memory_space=pltpu.MemorySpace.SMEM),
            pl.BlockSpec((B, 1), lambda: (0, 0)),
            pl.BlockSpec((1, B), lambda: (0, 0)),
            pl.BlockSpec((BH, 1, D), lambda: (0, 0, 0)),
            pl.BlockSpec((BH, 1, D), lambda: (0, 0, 0)),
        ],
        out_specs=pl.BlockSpec(memory_space=pltpu.MemorySpace.HBM),
        out_shape=jax.ShapeDtypeStruct((T, D), F32),
        input_output_aliases={0: 0},
        scratch_shapes=[
            pltpu.VMEM((B, D), F32),
            pltpu.SemaphoreType.DMA,
        ],
    )(new_hidden, gnn_i32, gnn_col, gnn_row, u0, u1)


def kernel(hidden_states, memory_nodes, gnn_idx, rel_idx, neighbor_idx,
           group_rel_ids, W_cls, b_cls, W_q, b_q, W_nodes, b_nodes, W_gnn,
           b_gnn):
    nbr_flat = neighbor_idx.reshape(-1).astype(I32)
    cat_idx = jnp.concatenate([gnn_idx, rel_idx]).astype(I32)
    sb = np.arange(SB)
    p_ = np.arange(S)
    c_ = np.arange(NB * KG)
    j_ = np.arange(NB)
    expTBD = jnp.asarray(
        (sb[:, None] // M == np.arange(NB * KG)[None, :]), BF16)
    konehBD = jnp.asarray(
        (sb[:, None] // (G * M) == np.arange(NB * K)[None, :]), BF16)
    grpA = jnp.asarray(
        np.where(p_[:, None] // R == c_[None, :] // KG, p_[:, None] % R, -1),
        I32)
    bmaskC = jnp.asarray((sb[None, :] // S == j_[:, None]), F32)
    blk64C = jnp.asarray((p_[None, :] // R == j_[:, None]), F32)
    gmaskC = jnp.asarray((c_[None, :] // KG == j_[:, None]), F32)

    q, rel_prob = _k1(cat_idx, hidden_states, W_q, b_q.reshape(1, D), W_cls,
                      b_cls.reshape(1, R))
    q3 = q.reshape(B, 1, D)
    relp3 = rel_prob.reshape(B, 1, R)
    grp3 = group_rel_ids.reshape(B // NB, 1, NB * KG).astype(I32)
    wn = W_nodes.astype(BF16)
    wg = W_gnn.astype(BF16)
    bn = b_nodes.reshape(1, D)
    bg = b_gnn.reshape(1, D)

    embs_parts = [
        _sc_gather(nbr_flat[c * BH * S:(c + 1) * BH * S], memory_nodes)
        for c in range(NSPLIT)
    ]
    rows_parts = []
    new_hidden = None
    for c in range(NSPLIT):
        rows_c, new_hidden = _k2_chunk(
            c, embs_parts[c], q3[c * BH:(c + 1) * BH],
            relp3[c * BH:(c + 1) * BH],
            grp3[c * NSTEP:(c + 1) * NSTEP], wn, bn, wg, bg, expTBD,
            konehBD, grpA, bmaskC, blk64C, gmaskC, hidden_states, new_hidden)
        rows_parts.append(rows_c)

    gnn_i32 = gnn_idx.astype(I32)
    return _k3(new_hidden, gnn_i32, gnn_i32.reshape(B, 1),
               gnn_i32.reshape(1, B), rows_parts[0], rows_parts[1])
